# baseline (speedup 1.0000x reference)
import functools

import jax
import jax.numpy as jnp
from jax import lax
from jax.experimental import pallas as pl
from jax.experimental.pallas import tpu as pltpu
from jax.experimental.pallas import tpu_sc as plsc

N = 10000
D = 256
E = 160000
NPAD = 10240
EPAD = 163840
NC = 2
NS = 16
HALF = NPAD // NC
TRASH = 64
RPT = HALF // NS
ZPAD = NPAD - N

_MESH = plsc.VectorSubcoreMesh(core_axis_name="c", subcore_axis_name="s")


def _k1_body(dst2d_hbm, out_hbm, didx, ones_v, zbuf, degsp):
    c = lax.axis_index("c")
    s = lax.axis_index("s")
    wid = c * NS + s

    def _zero(i, carry):
        zbuf[pl.ds(i * 16, 16)] = jnp.zeros((16,), jnp.float32)
        return carry

    lax.fori_loop(0, (NPAD // NS) // 16, _zero, 0)
    pltpu.sync_copy(zbuf, degsp.at[pl.ds(s * (NPAD // NS), NPAD // NS)])
    for k in range(8):
        ones_v[pl.ds(k * 16, 16)] = jnp.ones((16,), jnp.float32)
    pltpu.sync_copy(dst2d_hbm.at[pl.ds(wid * 40, 40)], didx)
    plsc.subcore_barrier()

    def _scat(j, carry):
        pltpu.sync_copy(ones_v, degsp.at[didx.at[j]], add=True)
        return carry

    lax.fori_loop(0, 40, _scat, 0)
    plsc.subcore_barrier()
    pltpu.sync_copy(degsp.at[pl.ds(s * (NPAD // NS), NPAD // NS)], zbuf)
    pltpu.sync_copy(zbuf, out_hbm.at[pl.ds(c * NPAD + s * (NPAD // NS), NPAD // NS)])


_k1 = pl.kernel(
    _k1_body,
    out_type=jax.ShapeDtypeStruct((NC * NPAD,), jnp.float32),
    mesh=_MESH,
    scratch_types=[
        pltpu.VMEM((40, 128), jnp.int32),
        pltpu.VMEM((128,), jnp.float32),
        pltpu.VMEM((NPAD // NS,), jnp.float32),
        pltpu.VMEM_SHARED((NPAD,), jnp.float32),
    ],
)


def _k2a_body(hn_ref, w_ref, y_ref):
    i = pl.program_id(0)
    y = jnp.dot(hn_ref[...], w_ref[...], preferred_element_type=jnp.float32)
    row = i * 512 + lax.broadcasted_iota(jnp.int32, (512, 1), 0)
    y_ref[...] = jnp.where(row < N, y, 0.0)


def _k2a(hn, W):
    nblk = NPAD // 512
    return pl.pallas_call(
        _k2a_body,
        grid=(nblk, 2),
        in_specs=[
            pl.BlockSpec((512, D), lambda i, j: (i, 0)),
            pl.BlockSpec((D, DH), lambda i, j: (0, j)),
        ],
        out_specs=pl.BlockSpec((512, DH), lambda i, j: (j * nblk + i, 0)),
        out_shape=jax.ShapeDtypeStruct((2 * NPAD, DH), jnp.float32),
    )(hn, W)


def _k2b_body(y_ref, degt_ref, z_ref):
    dd = degt_ref[...]
    deg = dd[:, 0] + dd[:, 1] + 1.0
    z_ref[...] = y_ref[...] * lax.rsqrt(deg)[:, None]


def _k2b(y2, degt):
    nblk = NPAD // 512
    return pl.pallas_call(
        _k2b_body,
        grid=(nblk, 2),
        in_specs=[
            pl.BlockSpec((512, DH), lambda i, j: (j * nblk + i, 0)),
            pl.BlockSpec((512, NC), lambda i, j: (i, 0)),
        ],
        out_specs=pl.BlockSpec((512, DH), lambda i, j: (j * nblk + i, 0)),
        out_shape=jax.ShapeDtypeStruct((2 * NPAD, DH), jnp.float32),
    )(y2, degt)


DH = 128
GE = 64
EPT = EPAD // NS
NB = EPT // GE
NRING = 4


def _k3_body(z2_hbm, combo_hbm, out_hbm, *scr):
    cbuf = scr[0]
    sstage = scr[1:1 + NRING]
    dstage = scr[1 + NRING:1 + 2 * NRING]
    gbuf = scr[1 + 2 * NRING:1 + 3 * NRING]
    gsem = scr[1 + 3 * NRING:1 + 4 * NRING]
    acc_ref = scr[1 + 4 * NRING]
    c = lax.axis_index("c")
    s = lax.axis_index("s")
    zoff = c * NPAD

    npt = NPAD // NS
    for k in range(npt // GE):
        rb = s * npt + k * GE
        pltpu.sync_copy(z2_hbm.at[pl.ds(zoff + rb, GE)], gbuf[0])
        pltpu.sync_copy(gbuf[0], acc_ref.at[pl.ds(rb, GE)])

    pltpu.sync_copy(combo_hbm.at[pl.ds(s * EPT, EPT)], cbuf)
    plsc.subcore_barrier()

    def _fill(stage_s, stage_d, bn):
        for k in range(GE // 16):
            combo = cbuf[pl.ds(bn * GE + k * 16, 16)]
            stage_s[pl.ds(k * 16, 16)] = zoff + lax.bitwise_and(combo, 0x3FFF)
            stage_d[pl.ds(k * 16, 16)] = lax.shift_right_logical(combo, 14)

    for q in range(NRING):
        _fill(sstage[q], dstage[q], jnp.int32(q))
        pltpu.async_copy(z2_hbm.at[sstage[q]], gbuf[q], gsem[q])

    def _quad(j, carry):
        for q in range(NRING):
            b = NRING * j + q
            pltpu.make_async_copy(z2_hbm.at[sstage[q]], gbuf[q], gsem[q]).wait()
            pltpu.sync_copy(gbuf[q], acc_ref.at[dstage[q]], add=True)
            _fill(sstage[q], dstage[q], jnp.minimum(b + NRING, NB - 1))
            pltpu.async_copy(z2_hbm.at[sstage[q]], gbuf[q], gsem[q])
        return carry

    lax.fori_loop(0, NB // NRING - 1, _quad, 0)
    for q in range(NRING):
        pltpu.make_async_copy(z2_hbm.at[sstage[q]], gbuf[q], gsem[q]).wait()
        pltpu.sync_copy(gbuf[q], acc_ref.at[dstage[q]], add=True)
        _fill(sstage[q], dstage[q], jnp.int32(NB - 1))
        pltpu.async_copy(z2_hbm.at[sstage[q]], gbuf[q], gsem[q])
    for q in range(NRING):
        pltpu.make_async_copy(z2_hbm.at[sstage[q]], gbuf[q], gsem[q]).wait()

    plsc.subcore_barrier()
    for k in range(npt // (2 * GE)):    plsc.subcore_barrier()
    for k in range(npt // (2 * GE)):
        rb = s * npt + k * 2 * GE
        pltpu.sync_copy(acc_ref.at[pl.ds(rb, GE)], gbuf[0])
        pltpu.sync_copy(acc_ref.at[pl.ds(rb + GE, GE)], gbuf[1])
        pltpu.sync_copy(gbuf[0], out_hbm.at[pl.ds(zoff + rb, GE)])
        pltpu.sync_copy(gbuf[1], out_hbm.at[pl.ds(zoff + rb + GE, GE)])


_k3 = pl.kernel(
    _k3_body,
    out_type=jax.ShapeDtypeStruct((2 * NPAD, DH), jnp.float32),
    mesh=_MESH,
    scratch_types=(
        [pltpu.VMEM((EPT,), jnp.int32)]
        + [pltpu.VMEM((GE,), jnp.int32) for _ in range(2 * NRING)]
        + [pltpu.VMEM((GE, DH), jnp.float32) for _ in range(NRING)]
        + [pltpu.SemaphoreType.DMA for _ in range(NRING)]
        + [pltpu.VMEM_SHARED((NPAD, DH), jnp.float32)]
    ),
)


def _k4_body(alo_ref, ahi_ref, degt_ref, b_ref, o_ref):
    dd = degt_ref[...]
    deg = dd[:, 0] + dd[:, 1] + 1.0
    dis = lax.rsqrt(deg)[:, None]
    bb = b_ref[...]
    lo = jnp.tanh(alo_ref[...] * dis + bb[:, :DH])
    hi = jnp.tanh(ahi_ref[...] * dis + bb[:, DH:])
    o_ref[...] = jnp.concatenate([lo, hi], axis=1)


def _k4(alo, ahi, degt, b2d):
    return pl.pallas_call(
        _k4_body,
        grid=(N // 1000,),
        in_specs=[
            pl.BlockSpec((1000, DH), lambda i: (i, 0)),
            pl.BlockSpec((1000, DH), lambda i: (i, 0)),
            pl.BlockSpec((1000, NC), lambda i: (i, 0)),
            pl.BlockSpec((1, D), lambda i: (0, 0)),
        ],
        out_specs=pl.BlockSpec((1000, D), lambda i: (i, 0)),
        out_shape=jax.ShapeDtypeStruct((N, D), jnp.float32),
    )(alo, ahi, degt, b2d)


def kernel(hn, edge_index, he, W, b):
    del he
    ei = edge_index.astype(jnp.int32)
    src, dst = ei[0], ei[1]
    npd = EPAD - E
    ar = jnp.arange(npd, dtype=jnp.int32)
    srcp = jnp.concatenate([src, N + ar % ZPAD])
    dstp = jnp.concatenate([dst, N + (ar * 7 + 13) % ZPAD])
    combo = srcp | (dstp << 14)
    dst2d = dstp.reshape(EPAD // 128, 128)
    y2 = _k2a(hn, W)
    degt = _k1(dst2d).reshape(NC, NPAD).T
    z2 = _k2b(y2, degt)
    agg2 = _k3(z2, combo)
    return _k4(agg2[:N], agg2[NPAD:NPAD + N], degt[:N], b.reshape(1, D))


# speedup vs baseline: 1.1372x; 1.1372x over previous
import functools

import jax
import jax.numpy as jnp
from jax import lax
from jax.experimental import pallas as pl
from jax.experimental.pallas import tpu as pltpu
from jax.experimental.pallas import tpu_sc as plsc

N = 10000
D = 256
E = 160000
NPAD = 10240
EPAD = 163840
NC = 2
NS = 16
HALF = NPAD // NC
TRASH = 64
RPT = HALF // NS
ZPAD = NPAD - N

_MESH = plsc.VectorSubcoreMesh(core_axis_name="c", subcore_axis_name="s")


def _k1_body(dst2d_hbm, out_hbm, didx, ones_v, zbuf, degsp):
    c = lax.axis_index("c")
    s = lax.axis_index("s")
    wid = c * NS + s

    def _zero(i, carry):
        zbuf[pl.ds(i * 16, 16)] = jnp.zeros((16,), jnp.float32)
        return carry

    lax.fori_loop(0, (NPAD // NS) // 16, _zero, 0)
    pltpu.sync_copy(zbuf, degsp.at[pl.ds(s * (NPAD // NS), NPAD // NS)])
    for k in range(8):
        ones_v[pl.ds(k * 16, 16)] = jnp.ones((16,), jnp.float32)
    pltpu.sync_copy(dst2d_hbm.at[pl.ds(wid * 40, 40)], didx)
    plsc.subcore_barrier()

    def _scat(j, carry):
        pltpu.sync_copy(ones_v, degsp.at[didx.at[j]], add=True)
        return carry

    lax.fori_loop(0, 40, _scat, 0)
    plsc.subcore_barrier()
    pltpu.sync_copy(degsp.at[pl.ds(s * (NPAD // NS), NPAD // NS)], zbuf)
    pltpu.sync_copy(zbuf, out_hbm.at[pl.ds(c * NPAD + s * (NPAD // NS), NPAD // NS)])


_k1 = pl.kernel(
    _k1_body,
    out_type=jax.ShapeDtypeStruct((NC * NPAD,), jnp.float32),
    mesh=_MESH,
    scratch_types=[
        pltpu.VMEM((40, 128), jnp.int32),
        pltpu.VMEM((128,), jnp.float32),
        pltpu.VMEM((NPAD // NS,), jnp.float32),
        pltpu.VMEM_SHARED((NPAD,), jnp.float32),
    ],
)


def _k2_body(hn_ref, w_ref, degp_ref, z_ref):
    i = pl.program_id(0)
    dd = degp_ref[...]
    deg = dd[0] + dd[1] + 1.0
    dis = lax.rsqrt(deg)
    y = jnp.dot(hn_ref[...], w_ref[...], preferred_element_type=jnp.float32)
    row = i * 512 + lax.broadcasted_iota(jnp.int32, (512, 1), 0)
    z_ref[...] = jnp.where(row < N, y * dis[:, None], 0.0)


def _k2(hn, W, degp):
    nblk = NPAD // 512
    return pl.pallas_call(
        _k2_body,
        grid=(nblk, 2),
        in_specs=[
            pl.BlockSpec((512, D), lambda i, j: (i, 0)),
            pl.BlockSpec((D, DH), lambda i, j: (0, j)),
            pl.BlockSpec((NC, 512), lambda i, j: (0, i)),
        ],
        out_specs=pl.BlockSpec((512, DH), lambda i, j: (j * nblk + i, 0)),
        out_shape=jax.ShapeDtypeStruct((2 * NPAD, DH), jnp.float32),
    )(hn, W, degp)


DH = 128
GE = 64
EPT = EPAD // NS
NB = EPT // GE
NRING = 4


def _k3_body(z2_hbm, combo_hbm, out_hbm, *scr):
    cbuf = scr[0]
    sstage = scr[1:1 + NRING]
    dstage = scr[1 + NRING:1 + 2 * NRING]
    gbuf = scr[1 + 2 * NRING:1 + 3 * NRING]
    gsem = scr[1 + 3 * NRING:1 + 4 * NRING]
    acc_ref = scr[1 + 4 * NRING]
    c = lax.axis_index("c")
    s = lax.axis_index("s")
    zoff = c * NPAD

    npt = NPAD // NS
    for k in range(npt // GE):
        rb = s * npt + k * GE
        pltpu.sync_copy(z2_hbm.at[pl.ds(zoff + rb, GE)], gbuf[0])
        pltpu.sync_copy(gbuf[0], acc_ref.at[pl.ds(rb, GE)])

    pltpu.sync_copy(combo_hbm.at[pl.ds(s * EPT, EPT)], cbuf)
    plsc.subcore_barrier()

    def _fill(stage_s, stage_d, bn):
        for k in range(GE // 16):
            combo = cbuf[pl.ds(bn * GE + k * 16, 16)]
            stage_s[pl.ds(k * 16, 16)] = zoff + lax.bitwise_and(combo, 0x3FFF)
            stage_d[pl.ds(k * 16, 16)] = lax.shift_right_logical(combo, 14)

    for q in range(NRING):
        _fill(sstage[q], dstage[q], jnp.int32(q))
        pltpu.async_copy(z2_hbm.at[sstage[q]], gbuf[q], gsem[q])

    def _quad(j, carry):
        for q in range(NRING):
            b = NRING * j + q
            pltpu.make_async_copy(z2_hbm.at[sstage[q]], gbuf[q], gsem[q]).wait()
            pltpu.sync_copy(gbuf[q], acc_ref.at[dstage[q]], add=True)
            _fill(sstage[q], dstage[q], jnp.minimum(b + NRING, NB - 1))
            pltpu.async_copy(z2_hbm.at[sstage[q]], gbuf[q], gsem[q])
        return carry

    lax.fori_loop(0, NB // NRING - 1, _quad, 0)
    for q in range(NRING):
        pltpu.make_async_copy(z2_hbm.at[sstage[q]], gbuf[q], gsem[q]).wait()
        pltpu.sync_copy(gbuf[q], acc_ref.at[dstage[q]], add=True)
        _fill(sstage[q], dstage[q], jnp.int32(NB - 1))
        pltpu.async_copy(z2_hbm.at[sstage[q]], gbuf[q], gsem[q])
    for q in range(NRING):
        pltpu.make_async_copy(z2_hbm.at[sstage[q]], gbuf[q], gsem[q]).wait()

    plsc.subcore_barrier()
    for k in range(npt // (2 * GE)):    plsc.subcore_barrier()
    for k in range(npt // (2 * GE)):
        rb = s * npt + k * 2 * GE
        pltpu.sync_copy(acc_ref.at[pl.ds(rb, GE)], gbuf[0])
        pltpu.sync_copy(acc_ref.at[pl.ds(rb + GE, GE)], gbuf[1])
        pltpu.sync_copy(gbuf[0], out_hbm.at[pl.ds(zoff + rb, GE)])
        pltpu.sync_copy(gbuf[1], out_hbm.at[pl.ds(zoff + rb + GE, GE)])


_k3 = pl.kernel(
    _k3_body,
    out_type=jax.ShapeDtypeStruct((2 * NPAD, DH), jnp.float32),
    mesh=_MESH,
    scratch_types=(
        [pltpu.VMEM((EPT,), jnp.int32)]
        + [pltpu.VMEM((GE,), jnp.int32) for _ in range(2 * NRING)]
        + [pltpu.VMEM((GE, DH), jnp.float32) for _ in range(NRING)]
        + [pltpu.SemaphoreType.DMA for _ in range(NRING)]
        + [pltpu.VMEM_SHARED((NPAD, DH), jnp.float32)]
    ),
)


def _k4_body(alo_ref, ahi_ref, degt_ref, b_ref, o_ref):
    dd = degt_ref[...]
    deg = dd[:, 0] + dd[:, 1] + 1.0
    dis = lax.rsqrt(deg)[:, None]
    bb = b_ref[...]
    lo = jnp.tanh(alo_ref[...] * dis + bb[:, :DH])
    hi = jnp.tanh(ahi_ref[...] * dis + bb[:, DH:])
    o_ref[...] = jnp.concatenate([lo, hi], axis=1)


def _k4(alo, ahi, degt, b2d):
    return pl.pallas_call(
        _k4_body,
        grid=(N // 1000,),
        in_specs=[
            pl.BlockSpec((1000, DH), lambda i: (i, 0)),
            pl.BlockSpec((1000, DH), lambda i: (i, 0)),
            pl.BlockSpec((1000, NC), lambda i: (i, 0)),
            pl.BlockSpec((1, D), lambda i: (0, 0)),
        ],
        out_specs=pl.BlockSpec((1000, D), lambda i: (i, 0)),
        out_shape=jax.ShapeDtypeStruct((N, D), jnp.float32),
    )(alo, ahi, degt, b2d)


def kernel(hn, edge_index, he, W, b):
    del he
    ei = edge_index.astype(jnp.int32)
    src, dst = ei[0], ei[1]
    npd = EPAD - E
    ar = jnp.arange(npd, dtype=jnp.int32)
    srcp = jnp.concatenate([src, N + ar % ZPAD])
    dstp = jnp.concatenate([dst, N + (ar * 7 + 13) % ZPAD])
    combo = srcp | (dstp << 14)
    dst2d = dstp.reshape(EPAD // 128, 128)
    degp = _k1(dst2d).reshape(NC, NPAD)
    z2 = _k2(hn, W, degp)
    degt = degp.T
    agg2 = _k3(z2, combo)
    return _k4(agg2[:N], agg2[NPAD:NPAD + N], degt[:N], b.reshape(1, D))
